# baseline (speedup 1.0000x reference)
import functools

import jax
import jax.numpy as jnp
from jax import lax
from jax.experimental import pallas as pl
from jax.experimental.pallas import tpu as pltpu
from jax.experimental.pallas import tpu_sc as plsc

BATCH = 16384
UID_NUM = 1000000
UID_DIM = 64
GEN_DIM = 8
AGE_DIM = 16
OCC_DIM = 16
GEN_NUM, AGE_NUM, OCC_NUM = 3, 100, 500
FINAL = 128
NC, NS, L = 2, 16, 16
NW = NC * NS
BPW = BATCH // NW
CH = 128
APACK = 128
BU = 4096
NBLK = 245
PAIR_ROWS = NBLK * (BU // 2)
B0 = 76
SCBLK = 244 - B0
SC_CHUNKS = SCBLK * (2048 // CH)
CPT = SC_CHUNKS // NW
SC_ROW0 = B0 * 2048


def _tc_pairize(tabT, eye64):
    dn = (((0,), (0,)), ((), ()))

    def body(x_ref, e_ref, o_ref):
        lo = lax.dot_general(x_ref[:, :BU // 2], e_ref[...], dn,
                             preferred_element_type=jnp.float32)
        hi = lax.dot_general(x_ref[:, BU // 2:], e_ref[...], dn,
                             preferred_element_type=jnp.float32)
        o_ref[...] = jnp.concatenate([lo, hi], axis=1)

    def blk(j):
        return jnp.where(j == B0, NBLK - 1, j)

    return pl.pallas_call(
        body,
        grid=(B0 + 1,),
        in_specs=[pl.BlockSpec((UID_DIM, BU), lambda j: (0, blk(j))),
                  pl.BlockSpec((UID_DIM, UID_DIM), lambda j: (0, 0))],
        out_specs=pl.BlockSpec((BU // 2, FINAL), lambda j: (blk(j), 0)),
        out_shape=jax.ShapeDtypeStruct((PAIR_ROWS, FINAL), jnp.float32),
    )(tabT, eye64)


def _sc_pairize(tabT):
    mesh = plsc.VectorSubcoreMesh(core_axis_name="c", subcore_axis_name="s")

    @functools.partial(
        pl.kernel,
        mesh=mesh,
        compiler_params=pltpu.CompilerParams(use_tc_tiling_on_sc=True,
                                             needs_layout_passes=False),
        out_type=jax.ShapeDtypeStruct((PAIR_ROWS, FINAL), jnp.float32),
        scratch_types=[
            pltpu.VMEM((UID_DIM, CH), jnp.float32),
            pltpu.VMEM((UID_DIM, CH), jnp.float32),
            pltpu.VMEM((UID_DIM, CH), jnp.float32),
            pltpu.VMEM((UID_DIM, CH), jnp.float32),
            pltpu.VMEM((CH, FINAL), jnp.float32),
            pltpu.VMEM((CH, FINAL), jnp.float32),
            pltpu.SemaphoreType.DMA,
            pltpu.SemaphoreType.DMA,
            pltpu.SemaphoreType.DMA,
            pltpu.SemaphoreType.DMA,
        ],
    )
    def k(tab_hbm, out_hbm, colA, colB, colA2, colB2, rowsA, rowsB,
          sem1, sem2, semwA, semwB):
        wid = lax.axis_index("s") * NC + lax.axis_index("c")
        cbase = wid * CPT
        lanes = lax.iota(jnp.int32, L)

        def issue(ci, ca, cb, sem):
            gc = cbase + ci
            b = B0 + lax.shift_right_logical(gc, 4)
            u0 = b * BU + (gc & 15) * CH
            u0 = pl.multiple_of(u0, CH)
            u1 = u0 + BU // 2
            for f8 in range(UID_DIM // 8):
                pltpu.async_copy(
                    tab_hbm.at[pl.ds(f8 * 8, 8), pl.ds(u0, CH)],
                    ca.at[pl.ds(f8 * 8, 8), :], sem)
                pltpu.async_copy(
                    tab_hbm.at[pl.ds(f8 * 8, 8), pl.ds(u1, CH)],
                    cb.at[pl.ds(f8 * 8, 8), :], sem)

        def drain(sem):
            for _ in range(2 * (UID_DIM // 8)):
                pltpu.make_async_copy(
                    tab_hbm.at[pl.ds(0, 8), pl.ds(0, CH)],
                    colA.at[pl.ds(0, 8), :], sem).wait()

        def transpose_write(ci, ca, cb, rows, semw):
            @pl.loop(0, CH // L)
            def _(g):
                rv = g * L + lanes
                for w in range(UID_DIM):
                    wv = jnp.full((L,), w, jnp.int32)
                    plsc.store_scatter(rows, [rv, wv],
                                       plsc.load_gather(ca, [wv, rv]))
                    plsc.store_scatter(rows, [rv, wv + UID_DIM],
                                       plsc.load_gather(cb, [wv, rv]))
            gc = cbase + ci
            row0 = SC_ROW0 + gc * CH
            row0 = pl.multiple_of(row0, CH)
            pltpu.async_copy(rows, out_hbm.at[pl.ds(row0, CH)], semw)

        def drain_write(rows, semw):
            pltpu.make_async_copy(
                rows, out_hbm.at[pl.ds(SC_ROW0, CH)], semw).wait()

        issue(0, colA, colB, sem1)

        @pl.loop(0, CPT // 2)
        def _(t):
            ci0 = t * 2
            issue(ci0 + 1, colA2, colB2, sem2)
            drain(sem1)

            @pl.when(t > 0)
            def _():
                drain_write(rowsA, semwA)

            transpose_write(ci0, colA, colB, rowsA, semwA)

            @pl.when(t < CPT // 2 - 1)
            def _():
                issue(ci0 + 2, colA, colB, sem1)

            drain(sem2)

            @pl.when(t > 0)
            def _():
                drain_write(rowsB, semwB)

            transpose_write(ci0 + 1, colA2, colB2, rowsB, semwB)

        drain_write(rowsA, semwA)
        drain_write(rowsB, semwB)

    return k(tabT)


def _sc_pair_gather(sample, pairA, pairB):
    mesh = plsc.VectorSubcoreMesh(core_axis_name="c", subcore_axis_name="s")

    @functools.partial(
        pl.kernel,
        mesh=mesh,
        compiler_params=pltpu.CompilerParams(use_tc_tiling_on_sc=True,
                                             needs_layout_passes=False),
        out_type=(jax.ShapeDtypeStruct((BATCH, FINAL), jnp.float32),
                  jax.ShapeDtypeStruct((BATCH, FINAL), jnp.float32)),
        scratch_types=[
            pltpu.VMEM((BPW,), jnp.int32),
            pltpu.VMEM((BPW,), jnp.int32),
            pltpu.VMEM((CH, FINAL), jnp.float32),
            pltpu.VMEM((CH, FINAL), jnp.float32),
        ],
    )
    def k(sample_hbm, pa_hbm, pb_hbm, outa_hbm, outb_hbm,
          idx_v, kidx_v, rowsa_v, rowsb_v):
        wid = lax.axis_index("s") * NC + lax.axis_index("c")
        base = wid * BPW
        pltpu.sync_copy(sample_hbm.at[pl.ds(base, BPW)], idx_v)

        @pl.loop(0, BPW // L)
        def _(g):
            s = idx_v[pl.ds(g * L, L)]
            kidx_v[pl.ds(g * L, L)] = (
                lax.shift_left(lax.shift_right_logical(s, 12), 11)
                + (s & (BU // 2 - 1)))

        @pl.loop(0, BPW // CH)
        def _(c):
            cb = c * CH
            idx = kidx_v.at[pl.ds(cb, CH)]
            pltpu.sync_copy(pa_hbm.at[idx], rowsa_v)
            pltpu.sync_copy(rowsa_v, outa_hbm.at[pl.ds(base + cb, CH)])
            pltpu.sync_copy(pb_hbm.at[idx], rowsb_v)
            pltpu.sync_copy(rowsb_v, outb_hbm.at[pl.ds(base + cb, CH)])

    return k(sample, pairA, pairB)


def _sc_attr_pack(sample, gen16, age16, occ16, mg, ma, mo):
    mesh = plsc.VectorSubcoreMesh(core_axis_name="c", subcore_axis_name="s")

    @functools.partial(
        pl.kernel,
        mesh=mesh,
        compiler_params=pltpu.CompilerParams(use_tc_tiling_on_sc=True,
                                             needs_layout_passes=False),
        out_type=jax.ShapeDtypeStruct((BATCH, APACK), jnp.float32),
        scratch_types=[
            pltpu.VMEM((BPW,), jnp.int32),
            pltpu.VMEM((BPW,), jnp.int32),
            pltpu.VMEM((BPW,), jnp.int32),
            pltpu.VMEM((BPW,), jnp.int32),
            pltpu.VMEM((GEN_NUM, L), jnp.float32),
            pltpu.VMEM((AGE_NUM, L), jnp.float32),
            pltpu.VMEM((OCC_NUM, L), jnp.float32),
            pltpu.VMEM((CH, APACK), jnp.float32),
            pltpu.SemaphoreType.DMA,
        ],
    )
    def k(sample_hbm, gen_hbm, age_hbm, occ_hbm, mg_hbm, ma_hbm, mo_hbm,
          attr_out, idx_v, gi_v, ai_v, oi_v, genv, agev, occv, pack_v, sem):
        wid = lax.axis_index("s") * NC + lax.axis_index("c")
        base = wid * BPW
        pltpu.sync_copy(sample_hbm.at[pl.ds(base, BPW)], idx_v)
        mg_dma = pltpu.async_copy(mg_hbm.at[idx_v], gi_v, sem)
        pltpu.sync_copy(gen_hbm, genv)
        pltpu.sync_copy(age_hbm, agev)
        pltpu.sync_copy(occ_hbm, occv)
        mg_dma.wait()
        ma_dma = pltpu.async_copy(ma_hbm.at[idx_v], ai_v, sem)
        mo_dma = pltpu.async_copy(mo_hbm.at[idx_v], oi_v, sem)
        ma_dma.wait()
        mo_dma.wait()

        lanes = lax.iota(jnp.int32, L)

        @pl.loop(0, BPW // CH)
        def _(c):
            cb = c * CH

            @pl.loop(0, CH // L)
            def _(g):
                j = g * L + lanes
                off = cb + g * L
                gvec = gi_v[pl.ds(off, L)]
                avec = ai_v[pl.ds(off, L)]
                ovec = oi_v[pl.ds(off, L)]
                for w in range(L):
                    wv = jnp.full((L,), w, jnp.int32)
                    plsc.store_scatter(
                        pack_v, [j, wv],
                        plsc.load_gather(genv, [gvec, wv]))
                    plsc.store_scatter(
                        pack_v, [j, wv + L],
                        plsc.load_gather(agev, [avec, wv]))
                    plsc.store_scatter(
                        pack_v, [j, wv + 2 * L],
                        plsc.load_gather(occv, [ovec, wv]))

            pltpu.sync_copy(pack_v, attr_out.at[pl.ds(base + cb, CH)])

    return k(sample, gen16, age16, occ16, mg, ma, mo)


def _tc_dense(pa, pb, attr, samp2d, W2, Wa, b):
    BLK = 2048
    KA = 3 * L

    def body(fa_ref, fb_ref, a_ref, s_ref, w2_ref, wa_ref, b_ref, o_ref):
        dn = (((1,), (0,)), ((), ()))
        s = s_ref[...]
        from_tc = (s < B0 * BU) | (s >= (NBLK - 1) * BU)
        f = jnp.where(from_tc, fa_ref[...], fb_ref[...])
        half = lax.shift_right_logical(s, 11) & 1
        lane_half = lax.shift_right_logical(
            lax.broadcasted_iota(jnp.int32, (BLK, FINAL), 1), 6)
        x = jnp.where(lane_half == half, f, 0.0)
        acc = lax.dot_general(x, w2_ref[...], dn,
                              preferred_element_type=jnp.float32)
        acc += lax.dot_general(a_ref[:, :KA], wa_ref[...], dn,
                               preferred_element_type=jnp.float32)
        o_ref[...] = jnp.tanh(acc + b_ref[...])

    return pl.pallas_call(
        body,
        grid=(BATCH // BLK,),
        in_specs=[
            pl.BlockSpec((BLK, FINAL), lambda i: (i, 0)),
            pl.BlockSpec((BLK, FINAL), lambda i: (i, 0)),
            pl.BlockSpec((BLK, APACK), lambda i: (i, 0)),
            pl.BlockSpec((BLK, 1), lambda i: (i, 0)),
            pl.BlockSpec((FINAL, FINAL), lambda i: (0, 0)),
            pl.BlockSpec((KA, FINAL), lambda i: (0, 0)),
            pl.BlockSpec((1, FINAL), lambda i: (0, 0)),
        ],
        out_specs=pl.BlockSpec((BLK, FINAL), lambda i: (i, 0)),
        out_shape=jax.ShapeDtypeStruct((BATCH, FINAL), jnp.float32),
    )(pa, pb, attr, samp2d, W2, Wa, b.reshape(1, FINAL))


def kernel(sample, user_id_emb, gender_emb, age_emb, occupation_emb,
           map_gender, map_age, map_occupation, W, b):
    sample = sample.astype(jnp.int32)
    tabT = user_id_emb.T
    pairA = _tc_pairize(tabT, jnp.eye(UID_DIM, dtype=jnp.float32))
    pairB = _sc_pairize(tabT)
    gen16 = jnp.pad(gender_emb, ((0, 0), (0, L - GEN_DIM)))
    zeros8 = jnp.zeros((L - GEN_DIM, FINAL), jnp.float32)
    Wu = W[:UID_DIM]
    W2 = jnp.concatenate([Wu, Wu], axis=0)
    Wa = jnp.concatenate([
        W[UID_DIM:UID_DIM + GEN_DIM], zeros8,
        W[UID_DIM + GEN_DIM:],
    ], axis=0)
    attr = _sc_attr_pack(sample, gen16, age_emb, occupation_emb,
                         map_gender.astype(jnp.int32),
                         map_age.astype(jnp.int32),
                         map_occupation.astype(jnp.int32))
    fa, fb = _sc_pair_gather(sample, pairA, pairB)
    return _tc_dense(fa, fb, attr, sample.reshape(BATCH, 1), W2, Wa, b)


# speedup vs baseline: 2.7408x; 2.7408x over previous
import functools

import jax
import jax.numpy as jnp
from jax import lax
from jax.experimental import pallas as pl
from jax.experimental.pallas import tpu as pltpu
from jax.experimental.pallas import tpu_sc as plsc

BATCH = 16384
UID_NUM = 1000000
UID_DIM = 64
GEN_DIM = 8
AGE_DIM = 16
OCC_DIM = 16
GEN_NUM, AGE_NUM, OCC_NUM = 3, 100, 500
FINAL = 128
NC, NS, L = 2, 16, 16
NW = NC * NS
BPW = BATCH // NW
CH = 128
APACK = 128
BU = 4096
NBLK = 245
PAIR_ROWS = NBLK * (BU // 2)
B0 = 200
SCBLK = 244 - B0
SC_CHUNKS = SCBLK * (2048 // CH)
CPT = SC_CHUNKS // NW
SC_ROW0 = B0 * 2048


def _tc_pairize(tabT, eye64):
    dn = (((0,), (0,)), ((), ()))

    def body(x_ref, e_ref, o_ref):
        lo = lax.dot_general(x_ref[:, :BU // 2], e_ref[...], dn,
                             preferred_element_type=jnp.float32)
        hi = lax.dot_general(x_ref[:, BU // 2:], e_ref[...], dn,
                             preferred_element_type=jnp.float32)
        o_ref[...] = jnp.concatenate([lo, hi], axis=1)

    def blk(j):
        return jnp.where(j == B0, NBLK - 1, j)

    return pl.pallas_call(
        body,
        grid=(B0 + 1,),
        in_specs=[pl.BlockSpec((UID_DIM, BU), lambda j: (0, blk(j))),
                  pl.BlockSpec((UID_DIM, UID_DIM), lambda j: (0, 0))],
        out_specs=pl.BlockSpec((BU // 2, FINAL), lambda j: (blk(j), 0)),
        out_shape=jax.ShapeDtypeStruct((PAIR_ROWS, FINAL), jnp.float32),
    )(tabT, eye64)


def _sc_pairize(tabT):
    mesh = plsc.VectorSubcoreMesh(core_axis_name="c", subcore_axis_name="s")

    @functools.partial(
        pl.kernel,
        mesh=mesh,
        compiler_params=pltpu.CompilerParams(use_tc_tiling_on_sc=True,
                                             needs_layout_passes=False),
        out_type=jax.ShapeDtypeStruct((PAIR_ROWS, FINAL), jnp.float32),
        scratch_types=[
            pltpu.VMEM((UID_DIM, CH), jnp.float32),
            pltpu.VMEM((UID_DIM, CH), jnp.float32),
            pltpu.VMEM((UID_DIM, CH), jnp.float32),
            pltpu.VMEM((UID_DIM, CH), jnp.float32),
            pltpu.VMEM((CH, FINAL), jnp.float32),
            pltpu.VMEM((CH, FINAL), jnp.float32),
            pltpu.SemaphoreType.DMA,
            pltpu.SemaphoreType.DMA,
            pltpu.SemaphoreType.DMA,
            pltpu.SemaphoreType.DMA,
        ],
    )
    def k(tab_hbm, out_hbm, colA, colB, colA2, colB2, rowsA, rowsB,
          sem1, sem2, semwA, semwB):
        wid = lax.axis_index("s") * NC + lax.axis_index("c")
        cbase = wid * CPT
        lanes = lax.iota(jnp.int32, L)

        def issue(ci, ca, cb, sem):
            gc = cbase + ci
            b = B0 + lax.shift_right_logical(gc, 4)
            u0 = b * BU + (gc & 15) * CH
            u0 = pl.multiple_of(u0, CH)
            u1 = u0 + BU // 2
            for f8 in range(UID_DIM // 8):
                pltpu.async_copy(
                    tab_hbm.at[pl.ds(f8 * 8, 8), pl.ds(u0, CH)],
                    ca.at[pl.ds(f8 * 8, 8), :], sem)
                pltpu.async_copy(
                    tab_hbm.at[pl.ds(f8 * 8, 8), pl.ds(u1, CH)],
                    cb.at[pl.ds(f8 * 8, 8), :], sem)

        def drain(sem):
            for _ in range(2 * (UID_DIM // 8)):
                pltpu.make_async_copy(
                    tab_hbm.at[pl.ds(0, 8), pl.ds(0, CH)],
                    colA.at[pl.ds(0, 8), :], sem).wait()

        def transpose_write(ci, ca, cb, rows, semw):
            @pl.loop(0, CH // L)
            def _(g):
                rv = g * L + lanes
                for w in range(UID_DIM):
                    wv = jnp.full((L,), w, jnp.int32)
                    plsc.store_scatter(rows, [rv, wv],
                                       ca[w, pl.ds(g * L, L)])
                    plsc.store_scatter(rows, [rv, wv + UID_DIM],
                                       cb[w, pl.ds(g * L, L)])
            gc = cbase + ci
            row0 = SC_ROW0 + gc * CH
            row0 = pl.multiple_of(row0, CH)
            pltpu.async_copy(rows, out_hbm.at[pl.ds(row0, CH)], semw)

        def drain_write(rows, semw):
            pltpu.make_async_copy(
                rows, out_hbm.at[pl.ds(SC_ROW0, CH)], semw).wait()

        issue(0, colA, colB, sem1)

        @pl.loop(0, CPT // 2)
        def _(t):
            ci0 = t * 2
            issue(ci0 + 1, colA2, colB2, sem2)
            drain(sem1)

            @pl.when(t > 0)
            def _():
                drain_write(rowsA, semwA)

            transpose_write(ci0, colA, colB, rowsA, semwA)

            @pl.when(t < CPT // 2 - 1)
            def _():
                issue(ci0 + 2, colA, colB, sem1)

            drain(sem2)

            @pl.when(t > 0)
            def _():
                drain_write(rowsB, semwB)

            transpose_write(ci0 + 1, colA2, colB2, rowsB, semwB)

        drain_write(rowsA, semwA)
        drain_write(rowsB, semwB)

    return k(tabT)


def _sc_pair_gather(sample, pairA, pairB):
    mesh = plsc.VectorSubcoreMesh(core_axis_name="c", subcore_axis_name="s")

    @functools.partial(
        pl.kernel,
        mesh=mesh,
        compiler_params=pltpu.CompilerParams(use_tc_tiling_on_sc=True,
                                             needs_layout_passes=False),
        out_type=(jax.ShapeDtypeStruct((BATCH, FINAL), jnp.float32),
                  jax.ShapeDtypeStruct((BATCH, FINAL), jnp.float32)),
        scratch_types=[
            pltpu.VMEM((BPW,), jnp.int32),
            pltpu.VMEM((BPW,), jnp.int32),
            pltpu.VMEM((CH, FINAL), jnp.float32),
            pltpu.VMEM((CH, FINAL), jnp.float32),
        ],
    )
    def k(sample_hbm, pa_hbm, pb_hbm, outa_hbm, outb_hbm,
          idx_v, kidx_v, rowsa_v, rowsb_v):
        wid = lax.axis_index("s") * NC + lax.axis_index("c")
        base = wid * BPW
        pltpu.sync_copy(sample_hbm.at[pl.ds(base, BPW)], idx_v)

        @pl.loop(0, BPW // L)
        def _(g):
            s = idx_v[pl.ds(g * L, L)]
            kidx_v[pl.ds(g * L, L)] = (
                lax.shift_left(lax.shift_right_logical(s, 12), 11)
                + (s & (BU // 2 - 1)))

        @pl.loop(0, BPW // CH)
        def _(c):
            cb = c * CH
            idx = kidx_v.at[pl.ds(cb, CH)]
            pltpu.sync_copy(pa_hbm.at[idx], rowsa_v)
            pltpu.sync_copy(rowsa_v, outa_hbm.at[pl.ds(base + cb, CH)])
            pltpu.sync_copy(pb_hbm.at[idx], rowsb_v)
            pltpu.sync_copy(rowsb_v, outb_hbm.at[pl.ds(base + cb, CH)])

    return k(sample, pairA, pairB)


def _sc_attr_pack(sample, gen16, age16, occ16, mg, ma, mo):
    mesh = plsc.VectorSubcoreMesh(core_axis_name="c", subcore_axis_name="s")

    @functools.partial(
        pl.kernel,
        mesh=mesh,
        compiler_params=pltpu.CompilerParams(use_tc_tiling_on_sc=True,
                                             needs_layout_passes=False),
        out_type=jax.ShapeDtypeStruct((BATCH, APACK), jnp.float32),
        scratch_types=[
            pltpu.VMEM((BPW,), jnp.int32),
            pltpu.VMEM((BPW,), jnp.int32),
            pltpu.VMEM((BPW,), jnp.int32),
            pltpu.VMEM((BPW,), jnp.int32),
            pltpu.VMEM((GEN_NUM, L), jnp.float32),
            pltpu.VMEM((AGE_NUM, L), jnp.float32),
            pltpu.VMEM((OCC_NUM, L), jnp.float32),
            pltpu.VMEM((CH, APACK), jnp.float32),
            pltpu.SemaphoreType.DMA,
        ],
    )
    def k(sample_hbm, gen_hbm, age_hbm, occ_hbm, mg_hbm, ma_hbm, mo_hbm,
          attr_out, idx_v, gi_v, ai_v, oi_v, genv, agev, occv, pack_v, sem):
        wid = lax.axis_index("s") * NC + lax.axis_index("c")
        base = wid * BPW
        pltpu.sync_copy(sample_hbm.at[pl.ds(base, BPW)], idx_v)
        mg_dma = pltpu.async_copy(mg_hbm.at[idx_v], gi_v, sem)
        pltpu.sync_copy(gen_hbm, genv)
        pltpu.sync_copy(age_hbm, agev)
        pltpu.sync_copy(occ_hbm, occv)
        mg_dma.wait()
        ma_dma = pltpu.async_copy(ma_hbm.at[idx_v], ai_v, sem)
        mo_dma = pltpu.async_copy(mo_hbm.at[idx_v], oi_v, sem)
        ma_dma.wait()
        mo_dma.wait()

        lanes = lax.iota(jnp.int32, L)

        @pl.loop(0, BPW // CH)
        def _(c):
            cb = c * CH

            @pl.loop(0, CH // L)
            def _(g):
                j = g * L + lanes
                off = cb + g * L
                gvec = gi_v[pl.ds(off, L)]
                avec = ai_v[pl.ds(off, L)]
                ovec = oi_v[pl.ds(off, L)]
                for w in range(L):
                    wv = jnp.full((L,), w, jnp.int32)
                    plsc.store_scatter(
                        pack_v, [j, wv],
                        plsc.load_gather(genv, [gvec, wv]))
                    plsc.store_scatter(
                        pack_v, [j, wv + L],
                        plsc.load_gather(agev, [avec, wv]))
                    plsc.store_scatter(
                        pack_v, [j, wv + 2 * L],
                        plsc.load_gather(occv, [ovec, wv]))

            pltpu.sync_copy(pack_v, attr_out.at[pl.ds(base + cb, CH)])

    return k(sample, gen16, age16, occ16, mg, ma, mo)


def _tc_dense(pa, pb, attr, samp2d, W2, Wa, b):
    BLK = 2048
    KA = 3 * L

    def body(fa_ref, fb_ref, a_ref, s_ref, w2_ref, wa_ref, b_ref, o_ref):
        dn = (((1,), (0,)), ((), ()))
        s = s_ref[...]
        from_tc = (s < B0 * BU) | (s >= (NBLK - 1) * BU)
        f = jnp.where(from_tc, fa_ref[...], fb_ref[...])
        half = lax.shift_right_logical(s, 11) & 1
        lane_half = lax.shift_right_logical(
            lax.broadcasted_iota(jnp.int32, (BLK, FINAL), 1), 6)
        x = jnp.where(lane_half == half, f, 0.0)
        acc = lax.dot_general(x, w2_ref[...], dn,
                              preferred_element_type=jnp.float32)
        acc += lax.dot_general(a_ref[:, :KA], wa_ref[...], dn,
                               preferred_element_type=jnp.float32)
        o_ref[...] = jnp.tanh(acc + b_ref[...])

    return pl.pallas_call(
        body,
        grid=(BATCH // BLK,),
        in_specs=[
            pl.BlockSpec((BLK, FINAL), lambda i: (i, 0)),
            pl.BlockSpec((BLK, FINAL), lambda i: (i, 0)),
            pl.BlockSpec((BLK, APACK), lambda i: (i, 0)),
            pl.BlockSpec((BLK, 1), lambda i: (i, 0)),
            pl.BlockSpec((FINAL, FINAL), lambda i: (0, 0)),
            pl.BlockSpec((KA, FINAL), lambda i: (0, 0)),
            pl.BlockSpec((1, FINAL), lambda i: (0, 0)),
        ],
        out_specs=pl.BlockSpec((BLK, FINAL), lambda i: (i, 0)),
        out_shape=jax.ShapeDtypeStruct((BATCH, FINAL), jnp.float32),
    )(pa, pb, attr, samp2d, W2, Wa, b.reshape(1, FINAL))


def kernel(sample, user_id_emb, gender_emb, age_emb, occupation_emb,
           map_gender, map_age, map_occupation, W, b):
    sample = sample.astype(jnp.int32)
    tabT = user_id_emb.T
    pairA = _tc_pairize(tabT, jnp.eye(UID_DIM, dtype=jnp.float32))
    pairB = _sc_pairize(tabT)
    gen16 = jnp.pad(gender_emb, ((0, 0), (0, L - GEN_DIM)))
    zeros8 = jnp.zeros((L - GEN_DIM, FINAL), jnp.float32)
    Wu = W[:UID_DIM]
    W2 = jnp.concatenate([Wu, Wu], axis=0)
    Wa = jnp.concatenate([
        W[UID_DIM:UID_DIM + GEN_DIM], zeros8,
        W[UID_DIM + GEN_DIM:],
    ], axis=0)
    attr = _sc_attr_pack(sample, gen16, age_emb, occupation_emb,
                         map_gender.astype(jnp.int32),
                         map_age.astype(jnp.int32),
                         map_occupation.astype(jnp.int32))
    fa, fb = _sc_pair_gather(sample, pairA, pairB)
    return _tc_dense(fa, fb, attr, sample.reshape(BATCH, 1), W2, Wa, b)


# speedup vs baseline: 2.7680x; 1.0099x over previous
import functools

import jax
import jax.numpy as jnp
from jax import lax
from jax.experimental import pallas as pl
from jax.experimental.pallas import tpu as pltpu
from jax.experimental.pallas import tpu_sc as plsc

BATCH = 16384
UID_NUM = 1000000
UID_DIM = 64
GEN_DIM = 8
AGE_DIM = 16
OCC_DIM = 16
GEN_NUM, AGE_NUM, OCC_NUM = 3, 100, 500
FINAL = 128
NC, NS, L = 2, 16, 16
NW = NC * NS
BPW = BATCH // NW
CH = 128
APACK = 128
BU = 4096
NBLK = 245
PAIR_ROWS = NBLK * (BU // 2)
B0 = 200
SCBLK = 244 - B0
SC_CHUNKS = SCBLK * (2048 // CH)
CPT = SC_CHUNKS // NW
SC_ROW0 = B0 * 2048


def _tc_pairize(tabT, eye64):
    dn = (((0,), (0,)), ((), ()))

    def body(x_ref, e_ref, o_ref):
        lo = lax.dot_general(x_ref[:, :BU // 2], e_ref[...], dn,
                             preferred_element_type=jnp.float32)
        hi = lax.dot_general(x_ref[:, BU // 2:], e_ref[...], dn,
                             preferred_element_type=jnp.float32)
        o_ref[...] = jnp.concatenate([lo, hi], axis=1)

    def blk(j):
        return jnp.where(j == B0, NBLK - 1, j)

    return pl.pallas_call(
        body,
        grid=(B0 + 1,),
        in_specs=[pl.BlockSpec((UID_DIM, BU), lambda j: (0, blk(j))),
                  pl.BlockSpec((UID_DIM, UID_DIM), lambda j: (0, 0))],
        out_specs=pl.BlockSpec((BU // 2, FINAL), lambda j: (blk(j), 0)),
        out_shape=jax.ShapeDtypeStruct((PAIR_ROWS, FINAL), jnp.float32),
    )(tabT, eye64)


def _sc_pairize(tabT):
    mesh = plsc.VectorSubcoreMesh(core_axis_name="c", subcore_axis_name="s")

    @functools.partial(
        pl.kernel,
        mesh=mesh,
        compiler_params=pltpu.CompilerParams(use_tc_tiling_on_sc=True,
                                             needs_layout_passes=False),
        out_type=jax.ShapeDtypeStruct((PAIR_ROWS, FINAL), jnp.float32),
        scratch_types=[
            pltpu.VMEM((UID_DIM, CH), jnp.float32),
            pltpu.VMEM((UID_DIM, CH), jnp.float32),
            pltpu.VMEM((UID_DIM, CH), jnp.float32),
            pltpu.VMEM((UID_DIM, CH), jnp.float32),
            pltpu.VMEM((CH, FINAL), jnp.float32),
            pltpu.VMEM((CH, FINAL), jnp.float32),
            pltpu.SemaphoreType.DMA,
            pltpu.SemaphoreType.DMA,
            pltpu.SemaphoreType.DMA,
            pltpu.SemaphoreType.DMA,
        ],
    )
    def k(tab_hbm, out_hbm, colA, colB, colA2, colB2, rowsA, rowsB,
          sem1, sem2, semwA, semwB):
        wid = lax.axis_index("s") * NC + lax.axis_index("c")
        cbase = wid * CPT
        lanes = lax.iota(jnp.int32, L)

        def issue(ci, ca, cb, sem):
            gc = cbase + ci
            b = B0 + lax.shift_right_logical(gc, 4)
            u0 = b * BU + (gc & 15) * CH
            u0 = pl.multiple_of(u0, CH)
            u1 = u0 + BU // 2
            for f8 in range(UID_DIM // 8):
                pltpu.async_copy(
                    tab_hbm.at[pl.ds(f8 * 8, 8), pl.ds(u0, CH)],
                    ca.at[pl.ds(f8 * 8, 8), :], sem)
                pltpu.async_copy(
                    tab_hbm.at[pl.ds(f8 * 8, 8), pl.ds(u1, CH)],
                    cb.at[pl.ds(f8 * 8, 8), :], sem)

        def drain(sem):
            for _ in range(2 * (UID_DIM // 8)):
                pltpu.make_async_copy(
                    tab_hbm.at[pl.ds(0, 8), pl.ds(0, CH)],
                    colA.at[pl.ds(0, 8), :], sem).wait()

        def transpose_write(ci, ca, cb, rows, semw):
            @pl.loop(0, CH // L)
            def _(g):
                rv = g * L + lanes
                for w in range(UID_DIM):
                    wv = jnp.full((L,), w, jnp.int32)
                    plsc.store_scatter(rows, [rv, wv],
                                       ca[w, pl.ds(g * L, L)])
                    plsc.store_scatter(rows, [rv, wv + UID_DIM],
                                       cb[w, pl.ds(g * L, L)])
            gc = cbase + ci
            row0 = SC_ROW0 + gc * CH
            row0 = pl.multiple_of(row0, CH)
            pltpu.async_copy(rows, out_hbm.at[pl.ds(row0, CH)], semw)

        def drain_write(rows, semw):
            pltpu.make_async_copy(
                rows, out_hbm.at[pl.ds(SC_ROW0, CH)], semw).wait()

        issue(0, colA, colB, sem1)

        @pl.loop(0, CPT // 2)
        def _(t):
            ci0 = t * 2
            issue(ci0 + 1, colA2, colB2, sem2)
            drain(sem1)

            @pl.when(t > 0)
            def _():
                drain_write(rowsA, semwA)

            transpose_write(ci0, colA, colB, rowsA, semwA)

            @pl.when(t < CPT // 2 - 1)
            def _():
                issue(ci0 + 2, colA, colB, sem1)

            drain(sem2)

            @pl.when(t > 0)
            def _():
                drain_write(rowsB, semwB)

            transpose_write(ci0 + 1, colA2, colB2, rowsB, semwB)

        drain_write(rowsA, semwA)
        drain_write(rowsB, semwB)

    return k(tabT)


def _sc_pair_gather(sample, pairA, pairB):
    mesh = plsc.VectorSubcoreMesh(core_axis_name="c", subcore_axis_name="s")

    @functools.partial(
        pl.kernel,
        mesh=mesh,
        compiler_params=pltpu.CompilerParams(use_tc_tiling_on_sc=True,
                                             needs_layout_passes=False),
        out_type=(jax.ShapeDtypeStruct((BATCH, FINAL), jnp.float32),
                  jax.ShapeDtypeStruct((BATCH, FINAL), jnp.float32)),
        scratch_types=[
            pltpu.VMEM((BPW,), jnp.int32),
            pltpu.VMEM((BPW,), jnp.int32),
            pltpu.VMEM((CH, FINAL), jnp.float32),
            pltpu.VMEM((CH, FINAL), jnp.float32),
            pltpu.SemaphoreType.DMA,
            pltpu.SemaphoreType.DMA,
        ],
    )
    def k(sample_hbm, pa_hbm, pb_hbm, outa_hbm, outb_hbm,
          idx_v, kidx_v, rowsa_v, rowsb_v, semA, semB):
        wid = lax.axis_index("s") * NC + lax.axis_index("c")
        base = wid * BPW
        pltpu.sync_copy(sample_hbm.at[pl.ds(base, BPW)], idx_v)

        @pl.loop(0, BPW // L)
        def _(g):
            s = idx_v[pl.ds(g * L, L)]
            kidx_v[pl.ds(g * L, L)] = (
                lax.shift_left(lax.shift_right_logical(s, 12), 11)
                + (s & (BU // 2 - 1)))

        @pl.loop(0, BPW // CH)
        def _(c):
            cb = c * CH
            idx = kidx_v.at[pl.ds(cb, CH)]
            da = pltpu.async_copy(pa_hbm.at[idx], rowsa_v, semA)
            db = pltpu.async_copy(pb_hbm.at[idx], rowsb_v, semB)
            da.wait()
            pltpu.sync_copy(rowsa_v, outa_hbm.at[pl.ds(base + cb, CH)])
            db.wait()
            pltpu.sync_copy(rowsb_v, outb_hbm.at[pl.ds(base + cb, CH)])

    return k(sample, pairA, pairB)


def _sc_attr_pack(sample, gen16, age16, occ16, mg, ma, mo):
    mesh = plsc.VectorSubcoreMesh(core_axis_name="c", subcore_axis_name="s")

    @functools.partial(
        pl.kernel,
        mesh=mesh,
        compiler_params=pltpu.CompilerParams(use_tc_tiling_on_sc=True,
                                             needs_layout_passes=False),
        out_type=jax.ShapeDtypeStruct((BATCH, APACK), jnp.float32),
        scratch_types=[
            pltpu.VMEM((BPW,), jnp.int32),
            pltpu.VMEM((BPW,), jnp.int32),
            pltpu.VMEM((BPW,), jnp.int32),
            pltpu.VMEM((BPW,), jnp.int32),
            pltpu.VMEM((GEN_NUM, L), jnp.float32),
            pltpu.VMEM((AGE_NUM, L), jnp.float32),
            pltpu.VMEM((OCC_NUM, L), jnp.float32),
            pltpu.VMEM((CH, APACK), jnp.float32),
            pltpu.SemaphoreType.DMA,
        ],
    )
    def k(sample_hbm, gen_hbm, age_hbm, occ_hbm, mg_hbm, ma_hbm, mo_hbm,
          attr_out, idx_v, gi_v, ai_v, oi_v, genv, agev, occv, pack_v, sem):
        wid = lax.axis_index("s") * NC + lax.axis_index("c")
        base = wid * BPW
        pltpu.sync_copy(sample_hbm.at[pl.ds(base, BPW)], idx_v)
        mg_dma = pltpu.async_copy(mg_hbm.at[idx_v], gi_v, sem)
        pltpu.sync_copy(gen_hbm, genv)
        pltpu.sync_copy(age_hbm, agev)
        pltpu.sync_copy(occ_hbm, occv)
        mg_dma.wait()
        ma_dma = pltpu.async_copy(ma_hbm.at[idx_v], ai_v, sem)
        mo_dma = pltpu.async_copy(mo_hbm.at[idx_v], oi_v, sem)
        ma_dma.wait()
        mo_dma.wait()

        lanes = lax.iota(jnp.int32, L)

        @pl.loop(0, BPW // CH)
        def _(c):
            cb = c * CH

            @pl.loop(0, CH // L)
            def _(g):
                j = g * L + lanes
                off = cb + g * L
                gvec = gi_v[pl.ds(off, L)]
                avec = ai_v[pl.ds(off, L)]
                ovec = oi_v[pl.ds(off, L)]
                for w in range(L):
                    wv = jnp.full((L,), w, jnp.int32)
                    plsc.store_scatter(
                        pack_v, [j, wv],
                        plsc.load_gather(genv, [gvec, wv]))
                    plsc.store_scatter(
                        pack_v, [j, wv + L],
                        plsc.load_gather(agev, [avec, wv]))
                    plsc.store_scatter(
                        pack_v, [j, wv + 2 * L],
                        plsc.load_gather(occv, [ovec, wv]))

            pltpu.sync_copy(pack_v, attr_out.at[pl.ds(base + cb, CH)])

    return k(sample, gen16, age16, occ16, mg, ma, mo)


def _tc_dense(pa, pb, attr, samp2d, W2, Wa, b):
    BLK = 2048
    KA = 3 * L

    def body(fa_ref, fb_ref, a_ref, s_ref, w2_ref, wa_ref, b_ref, o_ref):
        dn = (((1,), (0,)), ((), ()))
        s = s_ref[...]
        from_tc = (s < B0 * BU) | (s >= (NBLK - 1) * BU)
        f = jnp.where(from_tc, fa_ref[...], fb_ref[...])
        half = lax.shift_right_logical(s, 11) & 1
        lane_half = lax.shift_right_logical(
            lax.broadcasted_iota(jnp.int32, (BLK, FINAL), 1), 6)
        x = jnp.where(lane_half == half, f, 0.0)
        acc = lax.dot_general(x, w2_ref[...], dn,
                              preferred_element_type=jnp.float32)
        acc += lax.dot_general(a_ref[:, :KA], wa_ref[...], dn,
                               preferred_element_type=jnp.float32)
        o_ref[...] = jnp.tanh(acc + b_ref[...])

    return pl.pallas_call(
        body,
        grid=(BATCH // BLK,),
        in_specs=[
            pl.BlockSpec((BLK, FINAL), lambda i: (i, 0)),
            pl.BlockSpec((BLK, FINAL), lambda i: (i, 0)),
            pl.BlockSpec((BLK, APACK), lambda i: (i, 0)),
            pl.BlockSpec((BLK, 1), lambda i: (i, 0)),
            pl.BlockSpec((FINAL, FINAL), lambda i: (0, 0)),
            pl.BlockSpec((KA, FINAL), lambda i: (0, 0)),
            pl.BlockSpec((1, FINAL), lambda i: (0, 0)),
        ],
        out_specs=pl.BlockSpec((BLK, FINAL), lambda i: (i, 0)),
        out_shape=jax.ShapeDtypeStruct((BATCH, FINAL), jnp.float32),
    )(pa, pb, attr, samp2d, W2, Wa, b.reshape(1, FINAL))


def kernel(sample, user_id_emb, gender_emb, age_emb, occupation_emb,
           map_gender, map_age, map_occupation, W, b):
    sample = sample.astype(jnp.int32)
    tabT = user_id_emb.T
    pairA = _tc_pairize(tabT, jnp.eye(UID_DIM, dtype=jnp.float32))
    pairB = _sc_pairize(tabT)
    gen16 = jnp.pad(gender_emb, ((0, 0), (0, L - GEN_DIM)))
    zeros8 = jnp.zeros((L - GEN_DIM, FINAL), jnp.float32)
    Wu = W[:UID_DIM]
    W2 = jnp.concatenate([Wu, Wu], axis=0)
    Wa = jnp.concatenate([
        W[UID_DIM:UID_DIM + GEN_DIM], zeros8,
        W[UID_DIM + GEN_DIM:],
    ], axis=0)
    attr = _sc_attr_pack(sample, gen16, age_emb, occupation_emb,
                         map_gender.astype(jnp.int32),
                         map_age.astype(jnp.int32),
                         map_occupation.astype(jnp.int32))
    fa, fb = _sc_pair_gather(sample, pairA, pairB)
    return _tc_dense(fa, fb, attr, sample.reshape(BATCH, 1), W2, Wa, b)
